# Initial kernel scaffold; baseline (speedup 1.0000x reference)
import jax
import jax.numpy as jnp
from jax import lax
from jax.experimental import pallas as pl
from jax.experimental.pallas import tpu as pltpu
from jax.experimental.pallas import tpu_sc as plsc

N = 10000
D = 128
E = 320000
ALPHA = 0.5

NC = 2
NS = 16
TILES = NC * NS
NPAD = 10016
ROWS_PER_TILE = NPAD // NS
CHUNK = 128
ECHUNKS = 79
EPT = ECHUNKS * CHUNK
EPAD = TILES * EPT

_f32 = jnp.float32
_i32 = jnp.int32


def _sc_mesh():
    return plsc.VectorSubcoreMesh(
        core_axis_name="c", subcore_axis_name="s", num_cores=NC, num_subcores=NS
    )


def _deg_body(dst_hbm, out_hbm, dst_v, deg_v):
    cid = lax.axis_index("c")
    sid = lax.axis_index("s")
    wid = sid * NC + cid
    pltpu.sync_copy(dst_hbm.at[wid], dst_v)

    zeros16 = jnp.zeros((16,), _f32)
    ones16 = jnp.full((16,), 1.0, _f32)

    def zbody(i, carry):
        deg_v[pl.ds(i * 16, 16)] = zeros16
        return carry

    lax.fori_loop(0, NPAD // 16, zbody, 0)

    def ebody(g, carry):
        d16 = dst_v[pl.ds(g * 16, 16)]
        plsc.addupdate_scatter(deg_v, [d16], ones16)
        return carry

    lax.fori_loop(0, EPT // 16, ebody, 0)
    pltpu.sync_copy(deg_v, out_hbm.at[wid])


def _deg_call(dst_flat):
    k = pl.kernel(
        _deg_body,
        out_type=jax.ShapeDtypeStruct((TILES, NPAD), _f32),
        mesh=_sc_mesh(),
        scratch_types=[
            pltpu.VMEM((EPT,), _i32),
            pltpu.VMEM((NPAD,), _f32),
        ],
    )
    return k(dst_flat)


def _prop_body(u_hbm, src_hbm, dst_hbm, z_hbm, out_hbm, src_v, dst_v, buf, acc, sem):
    cid = lax.axis_index("c")
    sid = lax.axis_index("s")
    wid = sid * NC + cid
    pltpu.sync_copy(src_hbm.at[wid], src_v)
    pltpu.sync_copy(dst_hbm.at[wid], dst_v)

    row0 = sid * ROWS_PER_TILE

    @pl.when(cid == 0)
    def _():
        pltpu.sync_copy(u_hbm.at[pl.ds(row0, ROWS_PER_TILE)],
                        acc.at[pl.ds(row0, ROWS_PER_TILE)])

    @pl.when(cid != 0)
    def _():
        pltpu.sync_copy(z_hbm.at[pl.ds(row0, ROWS_PER_TILE)],
                        acc.at[pl.ds(row0, ROWS_PER_TILE)])

    plsc.subcore_barrier()

    for j in range(ECHUNKS):
        pltpu.async_copy(u_hbm.at[src_v.at[j]], buf, sem).wait()
        pltpu.sync_copy(buf, acc.at[dst_v.at[j]], add=True)

    plsc.subcore_barrier()
    pltpu.sync_copy(acc.at[pl.ds(row0, ROWS_PER_TILE)],
                    out_hbm.at[pl.ds(cid * NPAD + row0, ROWS_PER_TILE)])


def _prop_call(u, src3, dst3, zpad):
    k = pl.kernel(
        _prop_body,
        out_type=jax.ShapeDtypeStruct((NC * NPAD, D), _f32),
        mesh=_sc_mesh(),
        scratch_types=[
            pltpu.VMEM((ECHUNKS, CHUNK), _i32),
            pltpu.VMEM((ECHUNKS, CHUNK), _i32),
            pltpu.VMEM((CHUNK, D), _f32),
            pltpu.VMEM_SHARED((NPAD, D), _f32),
            pltpu.SemaphoreType.DMA,
        ],
    )
    return k(u, src3, dst3, zpad)


def _prep_body(parts_ref, x_ref, dinv_ref, u_ref):
    parts = parts_ref[...]
    ones = jnp.ones((TILES, 1), _f32)
    deg = lax.dot_general(parts, ones, (((0,), (0,)), ((), ())),
                          preferred_element_type=_f32) + 1.0
    dinv = lax.rsqrt(deg)
    dinv_ref[...] = dinv
    u_ref[...] = x_ref[...] * dinv


def _prep_call(parts, x_pad):
    return pl.pallas_call(
        _prep_body,
        out_shape=(
            jax.ShapeDtypeStruct((NPAD, 1), _f32),
            jax.ShapeDtypeStruct((NPAD, D), _f32),
        ),
    )(parts, x_pad)


def _combine_body(p_ref, dinv_ref, x_ref, u_ref):
    s = p_ref[:NPAD, :] + p_ref[NPAD:, :]
    dinv = dinv_ref[...]
    out = (1.0 - ALPHA) * dinv * s + ALPHA * x_ref[...]
    u_ref[...] = dinv * out


def _combine_call(p, dinv, x_pad):
    return pl.pallas_call(
        _combine_body,
        out_shape=jax.ShapeDtypeStruct((NPAD, D), _f32),
    )(p, dinv, x_pad)


def _final_body(p_ref, dinv_ref, x_ref, w_ref, b_ref, y_ref):
    s = p_ref[:NPAD, :] + p_ref[NPAD:, :]
    out = (1.0 - ALPHA) * dinv_ref[...] * s + ALPHA * x_ref[...]
    h = jnp.maximum(out, 0.0)
    y = lax.dot_general(h, w_ref[...], (((1,), (1,)), ((), ())),
                        preferred_element_type=_f32)
    y_ref[...] = y + b_ref[...]


def _final_call(p, dinv, x_pad, W, b):
    return pl.pallas_call(
        _final_body,
        out_shape=jax.ShapeDtypeStruct((NPAD, D), _f32),
    )(p, dinv, x_pad, W, b)


def kernel(x, edge_index, W, b):
    src = edge_index[0].astype(_i32)
    dst = edge_index[1].astype(_i32)
    src_p = jnp.concatenate([src, jnp.zeros((EPAD - E,), _i32)])
    dst_p = jnp.concatenate([dst, jnp.full((EPAD - E,), N, _i32)])
    src3 = src_p.reshape(TILES, ECHUNKS, CHUNK)
    dst3 = dst_p.reshape(TILES, ECHUNKS, CHUNK)
    dst_flat = dst_p.reshape(TILES, EPT)

    x_pad = jnp.pad(x, ((0, NPAD - N), (0, 0)))
    zpad = jnp.zeros((NPAD, D), _f32)

    deg_parts = _deg_call(dst_flat)
    dinv, u0 = _prep_call(deg_parts, x_pad)
    p1 = _prop_call(u0, src3, dst3, zpad)
    u1 = _combine_call(p1, dinv, x_pad)
    p2 = _prop_call(u1, src3, dst3, zpad)
    y = _final_call(p2, dinv, x_pad, W, b.reshape(1, D))
    return y[:N]


# speedup vs baseline: 11.3319x; 11.3319x over previous
import jax
import jax.numpy as jnp
from jax import lax
from jax.experimental import pallas as pl
from jax.experimental.pallas import tpu as pltpu
from jax.experimental.pallas import tpu_sc as plsc

N = 10000
D = 128
E = 320000
ALPHA = 0.5

NC = 2
NS = 16
TILES = NC * NS
NPAD = 10112
ROWS_PER_TILE = NPAD // NS
CHUNK = 128
ECHUNKS = 79
EPT = ECHUNKS * CHUNK
EPAD = TILES * EPT

_f32 = jnp.float32
_i32 = jnp.int32


def _sc_mesh():
    return plsc.VectorSubcoreMesh(
        core_axis_name="c", subcore_axis_name="s", num_cores=NC, num_subcores=NS
    )


def _deg_body(dst_hbm, out_hbm, dst_v, deg_v):
    cid = lax.axis_index("c")
    sid = lax.axis_index("s")
    wid = sid * NC + cid
    pltpu.sync_copy(dst_hbm.at[wid], dst_v)

    zeros16 = jnp.zeros((16,), _f32)
    ones16 = jnp.full((16,), 1.0, _f32)

    def zbody(i, carry):
        deg_v[pl.ds(i * 16, 16)] = zeros16
        return carry

    lax.fori_loop(0, NPAD // 16, zbody, 0)

    def ebody(g, carry):
        d16 = dst_v[pl.ds(g * 16, 16)]
        plsc.addupdate_scatter(deg_v, [d16], ones16)
        return carry

    lax.fori_loop(0, EPT // 16, ebody, 0)
    pltpu.sync_copy(deg_v, out_hbm.at[wid])


def _deg_call(dst_flat):
    k = pl.kernel(
        _deg_body,
        out_type=jax.ShapeDtypeStruct((TILES, NPAD), _f32),
        mesh=_sc_mesh(),
        compiler_params=pltpu.CompilerParams(needs_layout_passes=False),
        scratch_types=[
            pltpu.VMEM((EPT,), _i32),
            pltpu.VMEM((NPAD,), _f32),
        ],
    )
    return k(dst_flat)


def _prop_body(u_hbm, src_hbm, dst_hbm, z_hbm, out_hbm, src_v, dst_v, buf, acc, sem):
    cid = lax.axis_index("c")
    sid = lax.axis_index("s")
    wid = sid * NC + cid
    pltpu.sync_copy(src_hbm.at[wid], src_v)
    pltpu.sync_copy(dst_hbm.at[wid], dst_v)

    row0 = sid * ROWS_PER_TILE

    @pl.when(cid == 0)
    def _():
        pltpu.sync_copy(u_hbm.at[pl.ds(row0, ROWS_PER_TILE)],
                        acc.at[pl.ds(row0, ROWS_PER_TILE)])

    @pl.when(cid != 0)
    def _():
        pltpu.sync_copy(z_hbm.at[pl.ds(row0, ROWS_PER_TILE)],
                        acc.at[pl.ds(row0, ROWS_PER_TILE)])

    plsc.subcore_barrier()

    for j in range(ECHUNKS):
        pltpu.async_copy(u_hbm.at[src_v.at[j]], buf, sem).wait()
        pltpu.sync_copy(buf, acc.at[dst_v.at[j]], add=True)

    plsc.subcore_barrier()
    pltpu.sync_copy(acc.at[pl.ds(row0, ROWS_PER_TILE)],
                    out_hbm.at[pl.ds(cid * NPAD + row0, ROWS_PER_TILE)])


def _prop_call(u, src3, dst3, zpad):
    k = pl.kernel(
        _prop_body,
        out_type=jax.ShapeDtypeStruct((NC * NPAD, D), _f32),
        mesh=_sc_mesh(),
        compiler_params=pltpu.CompilerParams(needs_layout_passes=False),
        scratch_types=[
            pltpu.VMEM((ECHUNKS, CHUNK), _i32),
            pltpu.VMEM((ECHUNKS, CHUNK), _i32),
            pltpu.VMEM((CHUNK, D), _f32),
            pltpu.VMEM_SHARED((NPAD, D), _f32),
            pltpu.SemaphoreType.DMA,
        ],
    )
    return k(u, src3, dst3, zpad)


def _prep_body(parts_ref, x_ref, dinv_ref, u_ref):
    parts = parts_ref[...]
    ones = jnp.ones((TILES, 1), _f32)
    deg = lax.dot_general(parts, ones, (((0,), (0,)), ((), ())),
                          preferred_element_type=_f32) + 1.0
    dinv = lax.rsqrt(deg)
    dinv_ref[...] = dinv
    u_ref[...] = x_ref[...] * dinv


def _prep_call(parts, x_pad):
    return pl.pallas_call(
        _prep_body,
        out_shape=(
            jax.ShapeDtypeStruct((NPAD, 1), _f32),
            jax.ShapeDtypeStruct((NPAD, D), _f32),
        ),
    )(parts, x_pad)


def _combine_body(p_ref, dinv_ref, x_ref, u_ref):
    s = p_ref[:NPAD, :] + p_ref[NPAD:, :]
    dinv = dinv_ref[...]
    out = (1.0 - ALPHA) * dinv * s + ALPHA * x_ref[...]
    u_ref[...] = dinv * out


def _combine_call(p, dinv, x_pad):
    return pl.pallas_call(
        _combine_body,
        out_shape=jax.ShapeDtypeStruct((NPAD, D), _f32),
    )(p, dinv, x_pad)


def _final_body(p_ref, dinv_ref, x_ref, w_ref, b_ref, y_ref):
    s = p_ref[:NPAD, :] + p_ref[NPAD:, :]
    out = (1.0 - ALPHA) * dinv_ref[...] * s + ALPHA * x_ref[...]
    h = jnp.maximum(out, 0.0)
    y = lax.dot_general(h, w_ref[...], (((1,), (1,)), ((), ())),
                        preferred_element_type=_f32)
    y_ref[...] = y + b_ref[...]


def _final_call(p, dinv, x_pad, W, b):
    return pl.pallas_call(
        _final_body,
        out_shape=jax.ShapeDtypeStruct((NPAD, D), _f32),
    )(p, dinv, x_pad, W, b)


def kernel(x, edge_index, W, b):
    src = edge_index[0].astype(_i32)
    dst = edge_index[1].astype(_i32)
    src_p = jnp.concatenate([src, jnp.zeros((EPAD - E,), _i32)])
    dst_p = jnp.concatenate([dst, jnp.full((EPAD - E,), N, _i32)])
    src3 = src_p.reshape(TILES, ECHUNKS, CHUNK)
    dst3 = dst_p.reshape(TILES, ECHUNKS, CHUNK)
    dst_flat = dst_p.reshape(TILES, EPT)

    x_pad = jnp.pad(x, ((0, NPAD - N), (0, 0)))
    zpad = jnp.zeros((NPAD, D), _f32)

    deg_parts = _deg_call(dst_flat)
    dinv, u0 = _prep_call(deg_parts, x_pad)
    p1 = _prop_call(u0, src3, dst3, zpad)
    u1 = _combine_call(p1, dinv, x_pad)
    p2 = _prop_call(u1, src3, dst3, zpad)
    y = _final_call(p2, dinv, x_pad, W, b.reshape(1, D))
    return y[:N]
